# baseline (speedup 1.0000x reference)
import functools

import jax
import jax.numpy as jnp
from jax import lax
from jax.experimental import pallas as pl
from jax.experimental.pallas import tpu as pltpu
from jax.experimental.pallas import tpu_sc as plsc

VOCAB = 100000
EMBED = 16
BATCH = 1024
TILE = 2048
NTILES = (VOCAB + TILE - 1) // TILE
PADV = NTILES * TILE
K = 8
CB = BATCH // K


def _gather_sc(table, idx):
    info = plsc.get_sparse_core_info()
    nc, ns = info.num_cores, info.num_subcores
    nw = nc * ns
    bpw = BATCH // nw
    mesh = plsc.VectorSubcoreMesh(core_axis_name="c", subcore_axis_name="s")

    @functools.partial(
        pl.kernel,
        mesh=mesh,
        compiler_params=pltpu.CompilerParams(use_tc_tiling_on_sc=False),
        out_type=jax.ShapeDtypeStruct((BATCH, EMBED), jnp.float32),
        scratch_types=[
            pltpu.VMEM((bpw,), jnp.int32),
            pltpu.VMEM((bpw, EMBED), jnp.float32),
            pltpu.SemaphoreType.DMA,
        ],
    )
    def gk(table_hbm, idx_hbm, out_hbm, idx_v, rows_v, sem):
        wid = lax.axis_index("s") * nc + lax.axis_index("c")
        base = wid * bpw
        pltpu.sync_copy(idx_hbm.at[pl.ds(base, bpw)], idx_v)
        pltpu.async_copy(table_hbm.at[idx_v], rows_v, sem).wait()
        pltpu.sync_copy(rows_v, out_hbm.at[pl.ds(base, bpw)])

    return gk(table, idx)


def _fused(hb, wtp, b2p):
    def k(h_ref, w_ref, b_ref, o_ref, m_ref, s_ref, lse_ref):
        t = pl.program_id(0)
        p = t // NTILES
        j = lax.rem(t, NTILES)
        w = w_ref[:, pl.ds(j * TILE, TILE)]
        bcol = b_ref[:, pl.ds(j * TILE, TILE)]

        @pl.when(p < K)
        def _pass1():
            r0 = p * CB

            @pl.when(j == 0)
            def _():
                m_ref[pl.ds(r0, CB), :] = jnp.full((CB, 1), -1e30, jnp.float32)
                s_ref[pl.ds(r0, CB), :] = jnp.zeros((CB, 1), jnp.float32)

            hc = h_ref[pl.ds(r0, CB), :]
            logits = jnp.dot(hc, w, preferred_element_type=jnp.float32) + bcol
            m_old = m_ref[pl.ds(r0, CB), :]
            m_new = jnp.maximum(m_old, jnp.max(logits, axis=1, keepdims=True))
            s_ref[pl.ds(r0, CB), :] = s_ref[pl.ds(r0, CB), :] * jnp.exp(
                m_old - m_new
            ) + jnp.sum(jnp.exp(logits - m_new), axis=1, keepdims=True)
            m_ref[pl.ds(r0, CB), :] = m_new

            @pl.when(j == NTILES - 1)
            def _():
                lse_ref[pl.ds(r0, CB), :] = m_ref[pl.ds(r0, CB), :] + jnp.log(
                    s_ref[pl.ds(r0, CB), :]
                )

        @pl.when(p >= 1)
        def _pass2():
            r0 = (p - 1) * CB
            hc = h_ref[pl.ds(r0, CB), :]
            logits = jnp.dot(hc, w, preferred_element_type=jnp.float32)
            o_ref[...] = logits + (bcol - lse_ref[pl.ds(r0, CB), :])

    def out_map(t):
        p = t // NTILES
        return (jnp.maximum(p, 1) - 1, jnp.where(p >= 1, lax.rem(t, NTILES), 0))

    return pl.pallas_call(
        k,
        grid=((K + 1) * NTILES,),
        in_specs=[
            pl.BlockSpec((BATCH, EMBED), lambda t: (0, 0)),
            pl.BlockSpec((EMBED, PADV), lambda t: (0, 0)),
            pl.BlockSpec((1, PADV), lambda t: (0, 0)),
        ],
        out_specs=pl.BlockSpec((CB, TILE), out_map),
        out_shape=jax.ShapeDtypeStruct((BATCH, VOCAB), jnp.float32),
        scratch_shapes=[
            pltpu.VMEM((BATCH, 1), jnp.float32),
            pltpu.VMEM((BATCH, 1), jnp.float32),
            pltpu.VMEM((BATCH, 1), jnp.float32),
        ],
    )(hb, wtp, b2p)


def kernel(x, table, W, b):
    h = _gather_sc(table, x.astype(jnp.int32))
    hb = h.astype(jnp.bfloat16)
    wtp = jnp.concatenate(
        [W.T, jnp.zeros((EMBED, PADV - VOCAB), jnp.float32)], axis=1
    ).astype(jnp.bfloat16)
    b2p = jnp.concatenate(
        [b.reshape(1, VOCAB), jnp.full((1, PADV - VOCAB), -1e30, jnp.float32)],
        axis=1,
    )
    return _fused(hb, wtp, b2p)
